# baseline (speedup 1.0000x reference)
import jax
import jax.numpy as jnp
from jax import lax
from jax.experimental import pallas as pl
from jax.experimental.pallas import tpu as pltpu
from jax.experimental.pallas import tpu_sc as plsc

N = 10000
E = 320000
D = 128
OUT = 5
G = 20

NC = 2
NS = 16
NW = NC * NS
CHUNK = 128
CW = 80
EP = NW * CW * CHUNK
NP = 10240
DUMMY = N
RPT = NP // NS
GP = 32
BLK = 1024
NBW = 8
NBD = 2


def _mesh():
    return plsc.VectorSubcoreMesh(
        core_axis_name="c", subcore_axis_name="s",
        num_cores=NC, num_subcores=NS)


def _sc_degree(dstp4, ones_chunk, zeros_w):
    def body(dst_hbm, ones_hbm, zeros_hbm, out_hbm, didx_v, ones_v, acc_sh):
        c = lax.axis_index("c")
        s = lax.axis_index("s")
        wid = s * NC + c
        t0 = s * RPT
        pltpu.sync_copy(zeros_hbm, acc_sh.at[pl.ds(t0, RPT)])
        pltpu.sync_copy(ones_hbm, ones_v)
        pltpu.sync_copy(dst_hbm.at[wid], didx_v)
        plsc.subcore_barrier()

        def step(k, carry):
            pltpu.sync_copy(ones_v, acc_sh.at[didx_v.at[k]], add=True)
            return carry

        lax.fori_loop(0, CW, step, 0)
        plsc.subcore_barrier()
        pltpu.sync_copy(acc_sh.at[pl.ds(t0, RPT)],
                        out_hbm.at[c, pl.ds(t0, RPT)])

    return pl.kernel(
        body,
        out_type=jax.ShapeDtypeStruct((NC, NP, 16), jnp.float32),
        mesh=_mesh(),
        compiler_params=pltpu.CompilerParams(use_tc_tiling_on_sc=False),
        scratch_types=[
            pltpu.VMEM((CW, CHUNK), jnp.int32),
            pltpu.VMEM((CHUNK, 16), jnp.float32),
            pltpu.VMEM_SHARED((NP, 16), jnp.float32),
        ],
    )(dstp4, ones_chunk, zeros_w)


def _sc_scatter_full(table, srcp3, dstp3, zeros_w):
    def body(tab_hbm, src_hbm, dst_hbm, zeros_hbm, out_hbm,
             sidx_v, didx_g, rows_bufs, acc_sh, gsems, ssems):
        c = lax.axis_index("c")
        s = lax.axis_index("s")
        wid = s * NC + c
        t0 = s * RPT
        pltpu.sync_copy(zeros_hbm, acc_sh.at[pl.ds(t0, RPT)])
        pltpu.sync_copy(src_hbm.at[wid], sidx_v)
        plsc.subcore_barrier()

        def group(g, carry):
            base = g * NBD
            gds = []
            for b in range(NBD):
                gds.append(pltpu.async_copy(
                    tab_hbm.at[sidx_v.at[base + b]], rows_bufs[b], gsems[b]))
            pltpu.sync_copy(dst_hbm.at[wid, pl.ds(base, NBD)], didx_g)
            sds = []
            for b in range(NBD):
                gds[b].wait()
                sds.append(pltpu.async_copy(
                    rows_bufs[b], acc_sh.at[didx_g.at[b]], ssems[b],
                    add=True))
            for sd in sds:
                sd.wait()
            return carry

        lax.fori_loop(0, CW // NBD, group, 0)
        plsc.subcore_barrier()
        pltpu.sync_copy(acc_sh.at[pl.ds(t0, RPT)],
                        out_hbm.at[c, pl.ds(t0, RPT)])

    return pl.kernel(
        body,
        out_type=jax.ShapeDtypeStruct((NC, NP, D), jnp.float32),
        mesh=_mesh(),
        scratch_types=[
            pltpu.VMEM((CW, CHUNK), jnp.int32),
            pltpu.VMEM((NBD, CHUNK), jnp.int32),
            [pltpu.VMEM((CHUNK, D), jnp.float32) for _ in range(NBD)],
            pltpu.VMEM_SHARED((NP, D), jnp.float32),
            [pltpu.SemaphoreType.DMA for _ in range(NBD)],
            [pltpu.SemaphoreType.DMA for _ in range(NBD)],
        ],
    )(table, srcp3, dstp3, zeros_w)


def _sc_scatter_rows(table, srcp3, dstp4, zeros_w, width):
    def body(tab_hbm, src_hbm, dst_hbm, zeros_hbm, out_hbm,
             sidx_v, didx_v, rows_bufs, acc_sh, gsems, ssems):
        c = lax.axis_index("c")
        s = lax.axis_index("s")
        wid = s * NC + c
        t0 = s * RPT
        pltpu.sync_copy(zeros_hbm, acc_sh.at[pl.ds(t0, RPT)])
        pltpu.sync_copy(src_hbm.at[wid], sidx_v)
        pltpu.sync_copy(dst_hbm.at[wid], didx_v)
        plsc.subcore_barrier()

        def group(g, carry):
            base = g * NBW
            gds = []
            for b in range(NBW):
                gds.append(pltpu.async_copy(
                    tab_hbm.at[sidx_v.at[base + b]], rows_bufs[b], gsems[b]))
            sds = []
            for b in range(NBW):
                gds[b].wait()
                sds.append(pltpu.async_copy(
                    rows_bufs[b], acc_sh.at[didx_v.at[base + b]], ssems[b],
                    add=True))
            for sd in sds:
                sd.wait()
            return carry

        lax.fori_loop(0, CW // NBW, group, 0)
        plsc.subcore_barrier()
        pltpu.sync_copy(acc_sh.at[pl.ds(t0, RPT)],
                        out_hbm.at[c, pl.ds(t0, RPT)])

    return pl.kernel(
        body,
        out_type=jax.ShapeDtypeStruct((NC, NP, width), jnp.float32),
        mesh=_mesh(),
        compiler_params=pltpu.CompilerParams(
            use_tc_tiling_on_sc=False) if width < 128 else None,
        scratch_types=[
            pltpu.VMEM((CW, CHUNK), jnp.int32),
            pltpu.VMEM((CW, CHUNK), jnp.int32),
            [pltpu.VMEM((CHUNK, width), jnp.float32) for _ in range(NBW)],
            pltpu.VMEM_SHARED((NP, width), jnp.float32),
            [pltpu.SemaphoreType.DMA for _ in range(NBW)],
            [pltpu.SemaphoreType.DMA for _ in range(NBW)],
        ],
    )(table, srcp3, dstp4, zeros_w)


def _dinv_from(deg_blk):
    return lax.rsqrt(deg_blk[0][:, 0:1] + deg_blk[1][:, 0:1] + 1.0)


def _tc_h1(xp, W1, deg2):
    def body(x_ref, w_ref, deg_ref, out_ref):
        dinv = _dinv_from(deg_ref)
        h = jnp.dot(x_ref[...], w_ref[...],
                    preferred_element_type=jnp.float32)
        out_ref[...] = h * dinv

    return pl.pallas_call(
        body,
        grid=(NP // BLK,),
        in_specs=[
            pl.BlockSpec((BLK, D), lambda i: (i, 0)),
            pl.BlockSpec((D, D), lambda i: (0, 0)),
            pl.BlockSpec((NC, BLK, 16), lambda i: (0, i, 0)),
        ],
        out_specs=pl.BlockSpec((BLK, D), lambda i: (i, 0)),
        out_shape=jax.ShapeDtypeStruct((NP, D), jnp.float32),
    )(xp, W1, deg2)


def _tc_z(agg, h1p, deg2, b1r, W2p):
    def body(agg_ref, h1_ref, deg_ref, b1_ref, w2_ref, out_ref):
        dinv = _dinv_from(deg_ref)
        pre = dinv * (agg_ref[0] + agg_ref[1] + h1_ref[...]) + b1_ref[...]
        h = jnp.where(pre >= 0, pre, 0.01 * pre)
        z = jnp.dot(h, w2_ref[...], preferred_element_type=jnp.float32)
        out_ref[...] = z * dinv

    return pl.pallas_call(
        body,
        grid=(NP // BLK,),
        in_specs=[
            pl.BlockSpec((NC, BLK, D), lambda i: (0, i, 0)),
            pl.BlockSpec((BLK, D), lambda i: (i, 0)),
            pl.BlockSpec((NC, BLK, 16), lambda i: (0, i, 0)),
            pl.BlockSpec((1, D), lambda i: (0, 0)),
            pl.BlockSpec((D, 16), lambda i: (0, 0)),
        ],
        out_specs=pl.BlockSpec((BLK, 16), lambda i: (i, 0)),
        out_shape=jax.ShapeDtypeStruct((NP, 16), jnp.float32),
    )(agg, h1p, deg2, b1r, W2p)


def _tc_pool(agg2, zp, deg2, batch2d, b2p):
    def body(agg_ref, zp_ref, deg_ref, b_ref, b2_ref, out_ref):
        dinv = _dinv_from(deg_ref)
        a2 = dinv * (agg_ref[0] + agg_ref[1] + zp_ref[...])
        gids = lax.broadcasted_iota(jnp.int32, (GP, NP), 0)
        mask = (b_ref[...] == gids).astype(jnp.float32)
        gsum = jnp.dot(mask, a2, preferred_element_type=jnp.float32)
        cnt = jnp.sum(mask, axis=1, keepdims=True)
        out_ref[...] = gsum / jnp.maximum(cnt, 1.0) + b2_ref[...]

    return pl.pallas_call(
        body,
        out_shape=jax.ShapeDtypeStruct((GP, 16), jnp.float32),
    )(agg2, zp, deg2, batch2d, b2p)


def kernel(x, edge_index, batch, W1, b1, W2, b2):
    x = x.astype(jnp.float32)
    src = edge_index[0].astype(jnp.int32)
    dst = edge_index[1].astype(jnp.int32)
    pad = EP - E
    srcp3 = jnp.concatenate(
        [src, jnp.zeros((pad,), jnp.int32)]).reshape(NW, CW, CHUNK)
    dstp4 = jnp.concatenate(
        [dst, jnp.full((pad,), DUMMY, jnp.int32)]).reshape(NW, CW, CHUNK)
    xp = jnp.concatenate([x, jnp.zeros((NP - N, D), jnp.float32)])
    batch2d = jnp.concatenate(
        [batch.astype(jnp.int32), jnp.full((NP - N,), G, jnp.int32)]
    ).reshape(1, NP)
    ones_chunk = jnp.ones((CHUNK, 16), jnp.float32)
    zeros16 = jnp.zeros((RPT, 16), jnp.float32)
    zeros128 = jnp.zeros((RPT, D), jnp.float32)
    W2p = jnp.concatenate(
        [W2.astype(jnp.float32), jnp.zeros((D, 16 - OUT), jnp.float32)], axis=1)
    b1r = b1.astype(jnp.float32).reshape(1, D)
    b2p = jnp.concatenate(
        [b2.astype(jnp.float32), jnp.zeros((16 - OUT,), jnp.float32)]
    ).reshape(1, 16)

    deg2 = _sc_degree(dstp4, ones_chunk, zeros16)
    h1p = _tc_h1(xp, W1.astype(jnp.float32), deg2)
    agg = _sc_scatter_full(h1p, srcp3, dstp4, zeros128)
    zp = _tc_z(agg, h1p, deg2, b1r, W2p)
    agg2 = _sc_scatter_rows(zp, srcp3, dstp4, zeros16, 16)
    g = _tc_pool(agg2, zp, deg2, batch2d, b2p)
    g5 = g[:G, :OUT]
    return (g5[:, :2], g5[:, 2:4], g5[:, 4:5])


# speedup vs baseline: 1.2129x; 1.2129x over previous
import jax
import jax.numpy as jnp
from jax import lax
from jax.experimental import pallas as pl
from jax.experimental.pallas import tpu as pltpu
from jax.experimental.pallas import tpu_sc as plsc

N = 10000
E = 320000
D = 128
OUT = 5
G = 20

NC = 2
NS = 16
NW = NC * NS
CHUNK = 128
CW = 80
EP = NW * CW * CHUNK
NP = 10240
DUMMY = N
RPT = NP // NS
GP = 32
BLK = 1024
NBW = 8
NBD = 2
EPC = EP // CHUNK
CWT = EPC // NS
CWF = 120
FAST_CORE = 1


def _mesh():
    return plsc.VectorSubcoreMesh(
        core_axis_name="c", subcore_axis_name="s",
        num_cores=NC, num_subcores=NS)


def _sc_degree(dstp4, ones_chunk, zeros_w):
    def body(dst_hbm, ones_hbm, zeros_hbm, out_hbm, didx_v, ones_v, acc_sh):
        c = lax.axis_index("c")
        s = lax.axis_index("s")
        wid = s * NC + c
        t0 = s * RPT
        pltpu.sync_copy(zeros_hbm, acc_sh.at[pl.ds(t0, RPT)])
        pltpu.sync_copy(ones_hbm, ones_v)
        pltpu.sync_copy(dst_hbm.at[wid], didx_v)
        plsc.subcore_barrier()

        def step(k, carry):
            pltpu.sync_copy(ones_v, acc_sh.at[didx_v.at[k]], add=True)
            return carry

        lax.fori_loop(0, CW, step, 0)
        plsc.subcore_barrier()
        pltpu.sync_copy(acc_sh.at[pl.ds(t0, RPT)],
                        out_hbm.at[c, pl.ds(t0, RPT)])

    return pl.kernel(
        body,
        out_type=jax.ShapeDtypeStruct((NC, NP, 16), jnp.float32),
        mesh=_mesh(),
        compiler_params=pltpu.CompilerParams(use_tc_tiling_on_sc=False),
        scratch_types=[
            pltpu.VMEM((CW, CHUNK), jnp.int32),
            pltpu.VMEM((CHUNK, 16), jnp.float32),
            pltpu.VMEM_SHARED((NP, 16), jnp.float32),
        ],
    )(dstp4, ones_chunk, zeros_w)


def _sc_scatter_asym(table, src2d, dst2d, zeros_w, width, nbuf):
    def body(tab_hbm, src_hbm, dst_hbm, zeros_hbm, out_hbm,
             sidx_v, didx_g, rows_bufs, acc_sh, gsems, ssems):
        c = lax.axis_index("c")
        s = lax.axis_index("s")
        t0 = s * RPT
        pltpu.sync_copy(zeros_hbm, acc_sh.at[pl.ds(t0, RPT)])
        plsc.subcore_barrier()

        def run(nch, cb0):
            pltpu.sync_copy(src_hbm.at[pl.ds(cb0, nch)],
                            sidx_v.at[pl.ds(0, nch)])

            def group(g, carry):
                base = g * nbuf
                gds = []
                for b in range(nbuf):
                    gds.append(pltpu.async_copy(
                        tab_hbm.at[sidx_v.at[base + b]],
                        rows_bufs[b], gsems[b]))
                pltpu.sync_copy(dst_hbm.at[pl.ds(cb0 + base, nbuf)], didx_g)
                sds = []
                for b in range(nbuf):
                    gds[b].wait()
                    sds.append(pltpu.async_copy(
                        rows_bufs[b], acc_sh.at[didx_g.at[b]], ssems[b],
                        add=True))
                for sd in sds:
                    sd.wait()
                return carry

            lax.fori_loop(0, nch // nbuf, group, 0)

        @pl.when(c == FAST_CORE)
        def _fast():
            run(CWF, s * CWF)

        @pl.when(c != FAST_CORE)
        def _slow():
            run(CWT - CWF, NS * CWF + s * (CWT - CWF))

        plsc.subcore_barrier()
        pltpu.sync_copy(acc_sh.at[pl.ds(t0, RPT)],
                        out_hbm.at[c, pl.ds(t0, RPT)])

    return pl.kernel(
        body,
        out_type=jax.ShapeDtypeStruct((NC, NP, width), jnp.float32),
        mesh=_mesh(),
        compiler_params=pltpu.CompilerParams(
            use_tc_tiling_on_sc=False) if width < 128 else None,
        scratch_types=[
            pltpu.VMEM((CWF, CHUNK), jnp.int32),
            pltpu.VMEM((nbuf, CHUNK), jnp.int32),
            [pltpu.VMEM((CHUNK, width), jnp.float32) for _ in range(nbuf)],
            pltpu.VMEM_SHARED((NP, width), jnp.float32),
            [pltpu.SemaphoreType.DMA for _ in range(nbuf)],
            [pltpu.SemaphoreType.DMA for _ in range(nbuf)],
        ],
    )(table, src2d, dst2d, zeros_w)


def _dinv_from(deg_blk):
    return lax.rsqrt(deg_blk[0][:, 0:1] + deg_blk[1][:, 0:1] + 1.0)


def _tc_h1(xp, W1, deg2):
    def body(x_ref, w_ref, deg_ref, out_ref):
        dinv = _dinv_from(deg_ref)
        h = jnp.dot(x_ref[...], w_ref[...],
                    preferred_element_type=jnp.float32)
        out_ref[...] = h * dinv

    return pl.pallas_call(
        body,
        grid=(NP // BLK,),
        in_specs=[
            pl.BlockSpec((BLK, D), lambda i: (i, 0)),
            pl.BlockSpec((D, D), lambda i: (0, 0)),
            pl.BlockSpec((NC, BLK, 16), lambda i: (0, i, 0)),
        ],
        out_specs=pl.BlockSpec((BLK, D), lambda i: (i, 0)),
        out_shape=jax.ShapeDtypeStruct((NP, D), jnp.float32),
    )(xp, W1, deg2)


def _tc_z(agg, h1p, deg2, b1r, W2p):
    def body(agg_ref, h1_ref, deg_ref, b1_ref, w2_ref, out_ref):
        dinv = _dinv_from(deg_ref)
        pre = dinv * (agg_ref[0] + agg_ref[1] + h1_ref[...]) + b1_ref[...]
        h = jnp.where(pre >= 0, pre, 0.01 * pre)
        z = jnp.dot(h, w2_ref[...], preferred_element_type=jnp.float32)
        out_ref[...] = z * dinv

    return pl.pallas_call(
        body,
        grid=(NP // BLK,),
        in_specs=[
            pl.BlockSpec((NC, BLK, D), lambda i: (0, i, 0)),
            pl.BlockSpec((BLK, D), lambda i: (i, 0)),
            pl.BlockSpec((NC, BLK, 16), lambda i: (0, i, 0)),
            pl.BlockSpec((1, D), lambda i: (0, 0)),
            pl.BlockSpec((D, 16), lambda i: (0, 0)),
        ],
        out_specs=pl.BlockSpec((BLK, 16), lambda i: (i, 0)),
        out_shape=jax.ShapeDtypeStruct((NP, 16), jnp.float32),
    )(agg, h1p, deg2, b1r, W2p)


def _tc_pool(agg2, zp, deg2, batch2d, b2p):
    def body(agg_ref, zp_ref, deg_ref, b_ref, b2_ref, out_ref):
        dinv = _dinv_from(deg_ref)
        a2 = dinv * (agg_ref[0] + agg_ref[1] + zp_ref[...])
        gids = lax.broadcasted_iota(jnp.int32, (GP, NP), 0)
        mask = (b_ref[...] == gids).astype(jnp.float32)
        gsum = jnp.dot(mask, a2, preferred_element_type=jnp.float32)
        cnt = jnp.sum(mask, axis=1, keepdims=True)
        out_ref[...] = gsum / jnp.maximum(cnt, 1.0) + b2_ref[...]

    return pl.pallas_call(
        body,
        out_shape=jax.ShapeDtypeStruct((GP, 16), jnp.float32),
    )(agg2, zp, deg2, batch2d, b2p)


def kernel(x, edge_index, batch, W1, b1, W2, b2):
    x = x.astype(jnp.float32)
    src = edge_index[0].astype(jnp.int32)
    dst = edge_index[1].astype(jnp.int32)
    pad = EP - E
    srcp3 = jnp.concatenate(
        [src, jnp.zeros((pad,), jnp.int32)]).reshape(NW, CW, CHUNK)
    dstp4 = jnp.concatenate(
        [dst, jnp.full((pad,), DUMMY, jnp.int32)]).reshape(NW, CW, CHUNK)
    xp = jnp.concatenate([x, jnp.zeros((NP - N, D), jnp.float32)])
    batch2d = jnp.concatenate(
        [batch.astype(jnp.int32), jnp.full((NP - N,), G, jnp.int32)]
    ).reshape(1, NP)
    src2d = srcp3.reshape(EPC, CHUNK)
    dst2d = dstp4.reshape(EPC, CHUNK)
    ones_chunk = jnp.ones((CHUNK, 16), jnp.float32)
    zeros16 = jnp.zeros((RPT, 16), jnp.float32)
    zeros128 = jnp.zeros((RPT, D), jnp.float32)
    W2p = jnp.concatenate(
        [W2.astype(jnp.float32), jnp.zeros((D, 16 - OUT), jnp.float32)], axis=1)
    b1r = b1.astype(jnp.float32).reshape(1, D)
    b2p = jnp.concatenate(
        [b2.astype(jnp.float32), jnp.zeros((16 - OUT,), jnp.float32)]
    ).reshape(1, 16)

    deg2 = _sc_degree(dstp4, ones_chunk, zeros16)
    h1p = _tc_h1(xp, W1.astype(jnp.float32), deg2)
    agg = _sc_scatter_asym(h1p, src2d, dst2d, zeros128, D, NBD)
    zp = _tc_z(agg, h1p, deg2, b1r, W2p)
    agg2 = _sc_scatter_asym(zp, src2d, dst2d, zeros16, 16, NBW)
    g = _tc_pool(agg2, zp, deg2, batch2d, b2p)
    g5 = g[:G, :OUT]
    return (g5[:, :2], g5[:, 2:4], g5[:, 4:5])


# speedup vs baseline: 1.8368x; 1.5144x over previous
import jax
import jax.numpy as jnp
from jax import lax
from jax.experimental import pallas as pl
from jax.experimental.pallas import tpu as pltpu
from jax.experimental.pallas import tpu_sc as plsc

N = 10000
E = 320000
D = 128
OUT = 5
G = 20

NC = 2
NS = 16
NW = NC * NS
CHUNK = 128
CW = 80
EP = NW * CW * CHUNK
NP = 10240
DUMMY = N
RPT = NP // NS
GP = 32
BLK = 1024
NBW = 8
NBD = 2
EPC = EP // CHUNK
CWS = EPC // NS
NBS = 2
HW = D // NC


def _mesh():
    return plsc.VectorSubcoreMesh(
        core_axis_name="c", subcore_axis_name="s",
        num_cores=NC, num_subcores=NS)


def _sc_degree(dstp4, ones_chunk, zeros_w):
    def body(dst_hbm, ones_hbm, zeros_hbm, out_hbm, didx_v, ones_v, acc_sh):
        c = lax.axis_index("c")
        s = lax.axis_index("s")
        wid = s * NC + c
        t0 = s * RPT
        pltpu.sync_copy(zeros_hbm, acc_sh.at[pl.ds(t0, RPT)])
        pltpu.sync_copy(ones_hbm, ones_v)
        pltpu.sync_copy(dst_hbm.at[wid], didx_v)
        plsc.subcore_barrier()

        def step(k, carry):
            pltpu.sync_copy(ones_v, acc_sh.at[didx_v.at[k]], add=True)
            return carry

        lax.fori_loop(0, CW, step, 0)
        plsc.subcore_barrier()
        pltpu.sync_copy(acc_sh.at[pl.ds(t0, RPT)],
                        out_hbm.at[c, pl.ds(t0, RPT)])

    return pl.kernel(
        body,
        out_type=jax.ShapeDtypeStruct((NC, NP, 16), jnp.float32),
        mesh=_mesh(),
        compiler_params=pltpu.CompilerParams(use_tc_tiling_on_sc=False),
        scratch_types=[
            pltpu.VMEM((CW, CHUNK), jnp.int32),
            pltpu.VMEM((CHUNK, 16), jnp.float32),
            pltpu.VMEM_SHARED((NP, 16), jnp.float32),
        ],
    )(dstp4, ones_chunk, zeros_w)


def _sc_feat_spmem(tabs, src2d, dst2d, zeros_w):
    def body(tab_hbm, src_hbm, dst_hbm, zeros_hbm, out_hbm,
             sidx_v, didx_g, rows_bufs, tab_sh, acc_sh, gsems, ssems):
        c = lax.axis_index("c")
        s = lax.axis_index("s")
        t0 = s * RPT
        pltpu.sync_copy(tab_hbm.at[c, pl.ds(t0, RPT)],
                        tab_sh.at[pl.ds(t0, RPT)])
        pltpu.sync_copy(zeros_hbm, acc_sh.at[pl.ds(t0, RPT)])
        pltpu.sync_copy(src_hbm.at[pl.ds(s * CWS, CWS)], sidx_v)
        plsc.subcore_barrier()

        def group(g, carry):
            base = g * NBS
            gds = []
            for b in range(NBS):
                gds.append(pltpu.async_copy(
                    tab_sh.at[sidx_v.at[base + b]], rows_bufs[b], gsems[b]))
            pltpu.sync_copy(
                dst_hbm.at[pl.ds(s * CWS + base, NBS)], didx_g)
            sds = []
            for b in range(NBS):
                gds[b].wait()
                sds.append(pltpu.async_copy(
                    rows_bufs[b], acc_sh.at[didx_g.at[b]], ssems[b],
                    add=True))
            for sd in sds:
                sd.wait()
            return carry

        lax.fori_loop(0, CWS // NBS, group, 0)
        plsc.subcore_barrier()
        pltpu.sync_copy(acc_sh.at[pl.ds(t0, RPT)],
                        out_hbm.at[c, pl.ds(t0, RPT)])

    return pl.kernel(
        body,
        out_type=jax.ShapeDtypeStruct((NC, NP, HW), jnp.float32),
        mesh=_mesh(),
        compiler_params=pltpu.CompilerParams(use_tc_tiling_on_sc=False),
        scratch_types=[
            pltpu.VMEM((CWS, CHUNK), jnp.int32),
            pltpu.VMEM((NBS, CHUNK), jnp.int32),
            [pltpu.VMEM((CHUNK, HW), jnp.float32) for _ in range(NBS)],
            pltpu.VMEM_SHARED((NP, HW), jnp.float32),
            pltpu.VMEM_SHARED((NP, HW), jnp.float32),
            [pltpu.SemaphoreType.DMA for _ in range(NBS)],
            [pltpu.SemaphoreType.DMA for _ in range(NBS)],
        ],
    )(tabs, src2d, dst2d, zeros_w)


def _sc_edge_spmem(table, src2d, dst2d, zeros_w, width, nbuf):
    cpt = EPC // NW

    def body(tab_hbm, src_hbm, dst_hbm, zeros_hbm, out_hbm,
             sidx_v, didx_g, rows_bufs, tab_sh, acc_sh, gsems, ssems):
        c = lax.axis_index("c")
        s = lax.axis_index("s")
        wid = c * NS + s
        t0 = s * RPT
        pltpu.sync_copy(tab_hbm.at[pl.ds(t0, RPT)], tab_sh.at[pl.ds(t0, RPT)])
        pltpu.sync_copy(zeros_hbm, acc_sh.at[pl.ds(t0, RPT)])
        pltpu.sync_copy(src_hbm.at[pl.ds(wid * cpt, cpt)], sidx_v)
        plsc.subcore_barrier()

        def group(g, carry):
            base = g * nbuf
            gds = []
            for b in range(nbuf):
                gds.append(pltpu.async_copy(
                    tab_sh.at[sidx_v.at[base + b]], rows_bufs[b], gsems[b]))
            pltpu.sync_copy(
                dst_hbm.at[pl.ds(wid * cpt + base, nbuf)], didx_g)
            sds = []
            for b in range(nbuf):
                gds[b].wait()
                sds.append(pltpu.async_copy(
                    rows_bufs[b], acc_sh.at[didx_g.at[b]], ssems[b],
                    add=True))
            for sd in sds:
                sd.wait()
            return carry

        lax.fori_loop(0, cpt // nbuf, group, 0)
        plsc.subcore_barrier()
        pltpu.sync_copy(acc_sh.at[pl.ds(t0, RPT)],
                        out_hbm.at[c, pl.ds(t0, RPT)])

    return pl.kernel(
        body,
        out_type=jax.ShapeDtypeStruct((NC, NP, width), jnp.float32),
        mesh=_mesh(),
        compiler_params=pltpu.CompilerParams(use_tc_tiling_on_sc=False),
        scratch_types=[
            pltpu.VMEM((EPC // NW, CHUNK), jnp.int32),
            pltpu.VMEM((nbuf, CHUNK), jnp.int32),
            [pltpu.VMEM((CHUNK, width), jnp.float32) for _ in range(nbuf)],
            pltpu.VMEM_SHARED((NP, width), jnp.float32),
            pltpu.VMEM_SHARED((NP, width), jnp.float32),
            [pltpu.SemaphoreType.DMA for _ in range(nbuf)],
            [pltpu.SemaphoreType.DMA for _ in range(nbuf)],
        ],
    )(table, src2d, dst2d, zeros_w)


def _dinv_from(deg_blk):
    return lax.rsqrt(deg_blk[0][:, 0:1] + deg_blk[1][:, 0:1] + 1.0)


def _tc_h1(xp, W1, deg2):
    def body(x_ref, w_ref, deg_ref, out_ref):
        dinv = _dinv_from(deg_ref)
        h = jnp.dot(x_ref[...], w_ref[...],
                    preferred_element_type=jnp.float32)
        out_ref[...] = h * dinv

    return pl.pallas_call(
        body,
        grid=(NP // BLK,),
        in_specs=[
            pl.BlockSpec((BLK, D), lambda i: (i, 0)),
            pl.BlockSpec((D, D), lambda i: (0, 0)),
            pl.BlockSpec((NC, BLK, 16), lambda i: (0, i, 0)),
        ],
        out_specs=pl.BlockSpec((BLK, D), lambda i: (i, 0)),
        out_shape=jax.ShapeDtypeStruct((NP, D), jnp.float32),
    )(xp, W1, deg2)


def _tc_z(agg, h1p, deg2, b1r, W2p):
    def body(agg_ref, h1_ref, deg_ref, b1_ref, w2_ref, out_ref):
        dinv = _dinv_from(deg_ref)
        agg_full = jnp.concatenate([agg_ref[0], agg_ref[1]], axis=1)
        pre = dinv * (agg_full + h1_ref[...]) + b1_ref[...]
        h = jnp.where(pre >= 0, pre, 0.01 * pre)
        z = jnp.dot(h, w2_ref[...], preferred_element_type=jnp.float32)
        out_ref[...] = z * dinv

    return pl.pallas_call(
        body,
        grid=(NP // BLK,),
        in_specs=[
            pl.BlockSpec((NC, BLK, HW), lambda i: (0, i, 0)),
            pl.BlockSpec((BLK, D), lambda i: (i, 0)),
            pl.BlockSpec((NC, BLK, 16), lambda i: (0, i, 0)),
            pl.BlockSpec((1, D), lambda i: (0, 0)),
            pl.BlockSpec((D, 16), lambda i: (0, 0)),
        ],
        out_specs=pl.BlockSpec((BLK, 16), lambda i: (i, 0)),
        out_shape=jax.ShapeDtypeStruct((NP, 16), jnp.float32),
    )(agg, h1p, deg2, b1r, W2p)


def _tc_pool(agg2, zp, deg2, batch2d, b2p):
    def body(agg_ref, zp_ref, deg_ref, b_ref, b2_ref, out_ref):
        dinv = _dinv_from(deg_ref)
        a2 = dinv * (agg_ref[0] + agg_ref[1] + zp_ref[...])
        gids = lax.broadcasted_iota(jnp.int32, (GP, NP), 0)
        mask = (b_ref[...] == gids).astype(jnp.float32)
        gsum = jnp.dot(mask, a2, preferred_element_type=jnp.float32)
        cnt = jnp.sum(mask, axis=1, keepdims=True)
        out_ref[...] = gsum / jnp.maximum(cnt, 1.0) + b2_ref[...]

    return pl.pallas_call(
        body,
        out_shape=jax.ShapeDtypeStruct((GP, 16), jnp.float32),
    )(agg2, zp, deg2, batch2d, b2p)


def kernel(x, edge_index, batch, W1, b1, W2, b2):
    x = x.astype(jnp.float32)
    src = edge_index[0].astype(jnp.int32)
    dst = edge_index[1].astype(jnp.int32)
    pad = EP - E
    srcp3 = jnp.concatenate(
        [src, jnp.zeros((pad,), jnp.int32)]).reshape(NW, CW, CHUNK)
    dstp4 = jnp.concatenate(
        [dst, jnp.full((pad,), DUMMY, jnp.int32)]).reshape(NW, CW, CHUNK)
    xp = jnp.concatenate([x, jnp.zeros((NP - N, D), jnp.float32)])
    batch2d = jnp.concatenate(
        [batch.astype(jnp.int32), jnp.full((NP - N,), G, jnp.int32)]
    ).reshape(1, NP)
    src2d = srcp3.reshape(EPC, CHUNK)
    dst2d = dstp4.reshape(EPC, CHUNK)
    ones_chunk = jnp.ones((CHUNK, 16), jnp.float32)
    zeros16 = jnp.zeros((RPT, 16), jnp.float32)
    zeros64 = jnp.zeros((RPT, HW), jnp.float32)
    W2p = jnp.concatenate(
        [W2.astype(jnp.float32), jnp.zeros((D, 16 - OUT), jnp.float32)], axis=1)
    b1r = b1.astype(jnp.float32).reshape(1, D)
    b2p = jnp.concatenate(
        [b2.astype(jnp.float32), jnp.zeros((16 - OUT,), jnp.float32)]
    ).reshape(1, 16)

    deg2 = _sc_degree(dstp4, ones_chunk, zeros16)
    h1p = _tc_h1(xp, W1.astype(jnp.float32), deg2)
    tabs = h1p.reshape(NP, NC, HW).transpose(1, 0, 2)
    agg = _sc_feat_spmem(tabs, src2d, dst2d, zeros64)
    zp = _tc_z(agg, h1p, deg2, b1r, W2p)
    agg2 = _sc_edge_spmem(zp, src2d, dst2d, zeros16, 16, NBW)
    g = _tc_pool(agg2, zp, deg2, batch2d, b2p)
    g5 = g[:G, :OUT]
    return (g5[:, :2], g5[:, 2:4], g5[:, 4:5])


# speedup vs baseline: 1.8435x; 1.0036x over previous
import jax
import jax.numpy as jnp
from jax import lax
from jax.experimental import pallas as pl
from jax.experimental.pallas import tpu as pltpu
from jax.experimental.pallas import tpu_sc as plsc

N = 10000
E = 320000
D = 128
OUT = 5
G = 20

NC = 2
NS = 16
NW = NC * NS
CHUNK = 128
CW = 80
EP = NW * CW * CHUNK
NP = 10240
DUMMY = N
RPT = NP // NS
GP = 32
BLK = 1024
NBW = 8
NBD = 2
EPC = EP // CHUNK
CWS = EPC // NS
NBS = 2
HW = D // NC

assert CWS % NBS == 0 and (EPC // NW) % NBW == 0


def _mesh():
    return plsc.VectorSubcoreMesh(
        core_axis_name="c", subcore_axis_name="s",
        num_cores=NC, num_subcores=NS)


def _sc_degree(dstp4, ones_chunk, zeros_w):
    def body(dst_hbm, ones_hbm, zeros_hbm, out_hbm, didx_v, ones_v, acc_sh):
        c = lax.axis_index("c")
        s = lax.axis_index("s")
        wid = s * NC + c
        t0 = s * RPT
        pltpu.sync_copy(zeros_hbm, acc_sh.at[pl.ds(t0, RPT)])
        pltpu.sync_copy(ones_hbm, ones_v)
        pltpu.sync_copy(dst_hbm.at[wid], didx_v)
        plsc.subcore_barrier()

        def step(k, carry):
            pltpu.sync_copy(ones_v, acc_sh.at[didx_v.at[k]], add=True)
            return carry

        lax.fori_loop(0, CW, step, 0)
        plsc.subcore_barrier()
        pltpu.sync_copy(acc_sh.at[pl.ds(t0, RPT)],
                        out_hbm.at[c, pl.ds(t0, RPT)])

    return pl.kernel(
        body,
        out_type=jax.ShapeDtypeStruct((NC, NP, 16), jnp.float32),
        mesh=_mesh(),
        compiler_params=pltpu.CompilerParams(use_tc_tiling_on_sc=False),
        scratch_types=[
            pltpu.VMEM((CW, CHUNK), jnp.int32),
            pltpu.VMEM((CHUNK, 16), jnp.float32),
            pltpu.VMEM_SHARED((NP, 16), jnp.float32),
        ],
    )(dstp4, ones_chunk, zeros_w)


def _sc_feat_spmem(tabs, src2d, dst2d, zeros_w):
    def body(tab_hbm, src_hbm, dst_hbm, zeros_hbm, out_hbm,
             sidx_v, didx_g, rows_bufs, tab_sh, acc_sh, gsems, ssems):
        c = lax.axis_index("c")
        s = lax.axis_index("s")
        t0 = s * RPT
        pltpu.sync_copy(tab_hbm.at[c, pl.ds(t0, RPT)],
                        tab_sh.at[pl.ds(t0, RPT)])
        pltpu.sync_copy(zeros_hbm, acc_sh.at[pl.ds(t0, RPT)])
        pltpu.sync_copy(src_hbm.at[pl.ds(s * CWS, CWS)], sidx_v)
        plsc.subcore_barrier()

        def group(g, carry):
            base = g * NBS
            gds = []
            for b in range(NBS):
                gds.append(pltpu.async_copy(
                    tab_sh.at[sidx_v.at[base + b]], rows_bufs[b], gsems[b]))
            pltpu.sync_copy(
                dst_hbm.at[pl.ds(s * CWS + base, NBS)], didx_g)
            sds = []
            for b in range(NBS):
                gds[b].wait()
                sds.append(pltpu.async_copy(
                    rows_bufs[b], acc_sh.at[didx_g.at[b]], ssems[b],
                    add=True))
            for sd in sds:
                sd.wait()
            return carry

        lax.fori_loop(0, CWS // NBS, group, 0)
        plsc.subcore_barrier()
        pltpu.sync_copy(acc_sh.at[pl.ds(t0, RPT)],
                        out_hbm.at[c, pl.ds(t0, RPT)])

    return pl.kernel(
        body,
        out_type=jax.ShapeDtypeStruct((NC, NP, HW), jnp.float32),
        mesh=_mesh(),
        compiler_params=pltpu.CompilerParams(use_tc_tiling_on_sc=False),
        scratch_types=[
            pltpu.VMEM((CWS, CHUNK), jnp.int32),
            pltpu.VMEM((NBS, CHUNK), jnp.int32),
            [pltpu.VMEM((CHUNK, HW), jnp.float32) for _ in range(NBS)],
            pltpu.VMEM_SHARED((NP, HW), jnp.float32),
            pltpu.VMEM_SHARED((NP, HW), jnp.float32),
            [pltpu.SemaphoreType.DMA for _ in range(NBS)],
            [pltpu.SemaphoreType.DMA for _ in range(NBS)],
        ],
    )(tabs, src2d, dst2d, zeros_w)


def _sc_edge_spmem(table, src2d, dst2d, zeros_w, width, nbuf):
    cpt = EPC // NW

    def body(tab_hbm, src_hbm, dst_hbm, zeros_hbm, out_hbm,
             sidx_v, didx_g, rows_bufs, tab_sh, acc_sh, gsems, ssems):
        c = lax.axis_index("c")
        s = lax.axis_index("s")
        wid = c * NS + s
        t0 = s * RPT
        pltpu.sync_copy(tab_hbm.at[pl.ds(t0, RPT)], tab_sh.at[pl.ds(t0, RPT)])
        pltpu.sync_copy(zeros_hbm, acc_sh.at[pl.ds(t0, RPT)])
        pltpu.sync_copy(src_hbm.at[pl.ds(wid * cpt, cpt)], sidx_v)
        plsc.subcore_barrier()

        def group(g, carry):
            base = g * nbuf
            gds = []
            for b in range(nbuf):
                gds.append(pltpu.async_copy(
                    tab_sh.at[sidx_v.at[base + b]], rows_bufs[b], gsems[b]))
            pltpu.sync_copy(
                dst_hbm.at[pl.ds(wid * cpt + base, nbuf)], didx_g)
            sds = []
            for b in range(nbuf):
                gds[b].wait()
                sds.append(pltpu.async_copy(
                    rows_bufs[b], acc_sh.at[didx_g.at[b]], ssems[b],
                    add=True))
            for sd in sds:
                sd.wait()
            return carry

        lax.fori_loop(0, cpt // nbuf, group, 0)
        plsc.subcore_barrier()
        pltpu.sync_copy(acc_sh.at[pl.ds(t0, RPT)],
                        out_hbm.at[c, pl.ds(t0, RPT)])

    return pl.kernel(
        body,
        out_type=jax.ShapeDtypeStruct((NC, NP, width), jnp.float32),
        mesh=_mesh(),
        compiler_params=pltpu.CompilerParams(use_tc_tiling_on_sc=False),
        scratch_types=[
            pltpu.VMEM((EPC // NW, CHUNK), jnp.int32),
            pltpu.VMEM((nbuf, CHUNK), jnp.int32),
            [pltpu.VMEM((CHUNK, width), jnp.float32) for _ in range(nbuf)],
            pltpu.VMEM_SHARED((NP, width), jnp.float32),
            pltpu.VMEM_SHARED((NP, width), jnp.float32),
            [pltpu.SemaphoreType.DMA for _ in range(nbuf)],
            [pltpu.SemaphoreType.DMA for _ in range(nbuf)],
        ],
    )(table, src2d, dst2d, zeros_w)


def _dinv_from(deg_blk):
    return lax.rsqrt(deg_blk[0][:, 0:1] + deg_blk[1][:, 0:1] + 1.0)


def _tc_mm(xp, W1):
    def body(x_ref, w_ref, out_ref):
        out_ref[...] = jnp.dot(x_ref[...], w_ref[...],
                               preferred_element_type=jnp.float32)

    return pl.pallas_call(
        body,
        grid=(NP // BLK,),
        in_specs=[
            pl.BlockSpec((BLK, D), lambda i: (i, 0)),
            pl.BlockSpec((D, D), lambda i: (0, 0)),
        ],
        out_specs=pl.BlockSpec((BLK, D), lambda i: (i, 0)),
        out_shape=jax.ShapeDtypeStruct((NP, D), jnp.float32),
    )(xp, W1)


def _tc_scale(h1, deg2):
    def body(h_ref, deg_ref, out_ref):
        out_ref[...] = h_ref[...] * _dinv_from(deg_ref)

    return pl.pallas_call(
        body,
        grid=(NP // BLK,),
        in_specs=[
            pl.BlockSpec((BLK, D), lambda i: (i, 0)),
            pl.BlockSpec((NC, BLK, 16), lambda i: (0, i, 0)),
        ],
        out_specs=pl.BlockSpec((BLK, D), lambda i: (i, 0)),
        out_shape=jax.ShapeDtypeStruct((NP, D), jnp.float32),
    )(h1, deg2)


def _tc_z(agg, h1p, deg2, b1r, W2p):
    def body(agg_ref, h1_ref, deg_ref, b1_ref, w2_ref, out_ref):
        dinv = _dinv_from(deg_ref)
        agg_full = jnp.concatenate([agg_ref[0], agg_ref[1]], axis=1)
        pre = dinv * (agg_full + h1_ref[...]) + b1_ref[...]
        h = jnp.where(pre >= 0, pre, 0.01 * pre)
        z = jnp.dot(h, w2_ref[...], preferred_element_type=jnp.float32)
        out_ref[...] = z * dinv

    return pl.pallas_call(
        body,
        grid=(NP // BLK,),
        in_specs=[
            pl.BlockSpec((NC, BLK, HW), lambda i: (0, i, 0)),
            pl.BlockSpec((BLK, D), lambda i: (i, 0)),
            pl.BlockSpec((NC, BLK, 16), lambda i: (0, i, 0)),
            pl.BlockSpec((1, D), lambda i: (0, 0)),
            pl.BlockSpec((D, 16), lambda i: (0, 0)),
        ],
        out_specs=pl.BlockSpec((BLK, 16), lambda i: (i, 0)),
        out_shape=jax.ShapeDtypeStruct((NP, 16), jnp.float32),
    )(agg, h1p, deg2, b1r, W2p)


def _tc_pool(agg2, zp, deg2, batch2d, b2p):
    def body(agg_ref, zp_ref, deg_ref, b_ref, b2_ref, out_ref):
        dinv = _dinv_from(deg_ref)
        a2 = dinv * (agg_ref[0] + agg_ref[1] + zp_ref[...])
        gids = lax.broadcasted_iota(jnp.int32, (GP, NP), 0)
        mask = (b_ref[...] == gids).astype(jnp.float32)
        gsum = jnp.dot(mask, a2, preferred_element_type=jnp.float32)
        cnt = jnp.sum(mask, axis=1, keepdims=True)
        out_ref[...] = gsum / jnp.maximum(cnt, 1.0) + b2_ref[...]

    return pl.pallas_call(
        body,
        out_shape=jax.ShapeDtypeStruct((GP, 16), jnp.float32),
    )(agg2, zp, deg2, batch2d, b2p)


def kernel(x, edge_index, batch, W1, b1, W2, b2):
    x = x.astype(jnp.float32)
    src = edge_index[0].astype(jnp.int32)
    dst = edge_index[1].astype(jnp.int32)
    pad = EP - E
    srcp3 = jnp.concatenate(
        [src, jnp.zeros((pad,), jnp.int32)]).reshape(NW, CW, CHUNK)
    dstp4 = jnp.concatenate(
        [dst, jnp.full((pad,), DUMMY, jnp.int32)]).reshape(NW, CW, CHUNK)
    xp = jnp.concatenate([x, jnp.zeros((NP - N, D), jnp.float32)])
    batch2d = jnp.concatenate(
        [batch.astype(jnp.int32), jnp.full((NP - N,), G, jnp.int32)]
    ).reshape(1, NP)
    src2d = srcp3.reshape(EPC, CHUNK)
    dst2d = dstp4.reshape(EPC, CHUNK)
    ones_chunk = jnp.ones((CHUNK, 16), jnp.float32)
    zeros16 = jnp.zeros((RPT, 16), jnp.float32)
    zeros64 = jnp.zeros((RPT, HW), jnp.float32)
    W2p = jnp.concatenate(
        [W2.astype(jnp.float32), jnp.zeros((D, 16 - OUT), jnp.float32)], axis=1)
    b1r = b1.astype(jnp.float32).reshape(1, D)
    b2p = jnp.concatenate(
        [b2.astype(jnp.float32), jnp.zeros((16 - OUT,), jnp.float32)]
    ).reshape(1, 16)

    h1 = _tc_mm(xp, W1.astype(jnp.float32))
    deg2 = _sc_degree(dstp4, ones_chunk, zeros16)
    h1p = _tc_scale(h1, deg2)
    tabs = h1p.reshape(NP, NC, HW).transpose(1, 0, 2)
    agg = _sc_feat_spmem(tabs, src2d, dst2d, zeros64)
    zp = _tc_z(agg, h1p, deg2, b1r, W2p)
    agg2 = _sc_edge_spmem(zp, src2d, dst2d, zeros16, 16, NBW)
    g = _tc_pool(agg2, zp, deg2, batch2d, b2p)
    g5 = g[:G, :OUT]
    return (g5[:, :2], g5[:, 2:4], g5[:, 4:5])
